# baseline (speedup 1.0000x reference)
import functools

import jax
import jax.numpy as jnp
from jax import lax
from jax.experimental import pallas as pl
from jax.experimental.pallas import tpu as pltpu
from jax.experimental.pallas import tpu_sc as plsc

_B = 16384
_D = 64
_H = 128
_CH = 128
_P = 4


def _round_bits(x):
    return lax.bitcast_convert_type(
        x.astype(jnp.bfloat16).astype(jnp.float32), jnp.int32)


def _transpose_body(x0_ref, x1_ref, x2_ref, x3_ref, out_ref):
    wa = _round_bits(jnp.concatenate([x0_ref[...].T, x1_ref[...].T], axis=1))
    wb = _round_bits(jnp.concatenate([x2_ref[...].T, x3_ref[...].T], axis=1))
    out_ref[...] = wa | lax.shift_right_logical(wb, 16)


def _pack_geometry(n_rows, R):
    off = (n_rows // _P) // R * R
    grid = -(-(n_rows - (_P - 1) * off) // R)
    return off, grid


@functools.cache
def _transpose_fn(n_rows, R):
    off, grid = _pack_geometry(n_rows, R)
    ob = off // R
    return pl.pallas_call(
        _transpose_body,
        grid=(grid,),
        in_specs=[
            pl.BlockSpec((_D, R), lambda i, k=k: (0, i + k * ob))
            for k in range(_P)
        ],
        out_specs=pl.BlockSpec((R, _P * _D // 2), lambda i: (i, 0)),
        out_shape=jax.ShapeDtypeStruct((grid * R, _P * _D // 2), jnp.int32),
    )


@functools.cache
def _gather_fn(B, NC, NS, n_packed):
    NW = NC * NS
    b_per_w = B // NW
    n_ch = b_per_w // _CH
    mesh = plsc.VectorSubcoreMesh(core_axis_name="c", subcore_axis_name="s")

    @functools.partial(
        pl.kernel,
        out_type=jax.ShapeDtypeStruct((B, _P * _D // 2), jnp.int32),
        mesh=mesh,
        scratch_types=[
            pltpu.VMEM((n_ch, _CH), jnp.int32),
            pltpu.VMEM((n_ch, _CH, _P * _D // 2), jnp.int32),
            pltpu.SemaphoreType.DMA,
        ],
    )
    def gather_k(idx_hbm, tbl_hbm, x_out, idx_v, rows_v, sem):
        wid = lax.axis_index("s") * NC + lax.axis_index("c")
        base = wid * b_per_w
        pltpu.sync_copy(idx_hbm.at[pl.ds(wid * n_ch, n_ch)], idx_v)
        copies = [
            pltpu.async_copy(tbl_hbm.at[idx_v.at[c]], rows_v.at[c], sem)
            for c in range(n_ch)
        ]
        for c in range(n_ch):
            copies[c].wait()
            pltpu.sync_copy(rows_v.at[c],
                            x_out.at[pl.ds(base + c * _CH, _CH)])

    return gather_k


def _select_quarter(x_ref, sub):
    w = x_ref[...]
    hi = lax.bitcast_convert_type(w & jnp.int32(-65536), jnp.float32)
    lo = lax.bitcast_convert_type(w << 16, jnp.float32)
    a = jnp.where(sub == 0, hi[:, :_D], hi[:, _D:])
    b = jnp.where(sub == 2, lo[:, :_D], lo[:, _D:])
    return jnp.where(sub <= 1, a, b)


def _mlp_body(xu_ref, xi_ref, pu_ref, pi_ref,
              w1u_ref, w1i_ref, b1_ref, w2_ref, b2_ref, out_ref):
    ue = _select_quarter(xu_ref, pu_ref[...])
    ie = _select_quarter(xi_ref, pi_ref[...])
    h = (jnp.dot(ue, w1u_ref[...], preferred_element_type=jnp.float32)
         + jnp.dot(ie, w1i_ref[...], preferred_element_type=jnp.float32)
         + b1_ref[...])
    h = jnp.maximum(h, 0.0)
    out_ref[...] = (
        jnp.dot(h, w2_ref[...], preferred_element_type=jnp.float32) + b2_ref[...]
    )


@functools.cache
def _mlp_fn(B, D, H, blk):
    grid = B // blk
    return pl.pallas_call(
        _mlp_body,
        grid=(grid,),
        in_specs=[
            pl.BlockSpec((blk, _P * D // 2), lambda i: (i, 0)),
            pl.BlockSpec((blk, _P * D // 2), lambda i: (i, 0)),
            pl.BlockSpec((blk, 1), lambda i: (i, 0)),
            pl.BlockSpec((blk, 1), lambda i: (i, 0)),
            pl.BlockSpec((D, H), lambda i: (0, 0)),
            pl.BlockSpec((D, H), lambda i: (0, 0)),
            pl.BlockSpec((1, H), lambda i: (0, 0)),
            pl.BlockSpec((H, 1), lambda i: (0, 0)),
            pl.BlockSpec((1, 1), lambda i: (0, 0)),
        ],
        out_specs=pl.BlockSpec((blk, 1), lambda i: (i, 0)),
        out_shape=jax.ShapeDtypeStruct((B, 1), jnp.float32),
    )


def kernel(user, item, user_table, item_table, W1, b1, W2, b2):
    info = plsc.get_sparse_core_info()
    n = user_table.shape[0]
    R = 11904
    off, grid = _pack_geometry(n, R)
    tr = _transpose_fn(n, R)
    gather = _gather_fn(_B, info.num_cores, info.num_subcores, grid * R)

    def fold(i):
        sub = ((i >= off).astype(jnp.int32) + (i >= 2 * off).astype(jnp.int32)
               + (i >= 3 * off).astype(jnp.int32))
        return i - sub * off, sub

    uq, us = fold(user)
    iq, isel = fold(item)
    u2 = tr(user_table.T, user_table.T, user_table.T, user_table.T)
    xu = gather(uq.reshape(_B // _CH, _CH), u2)
    i2 = tr(item_table.T, item_table.T, item_table.T, item_table.T)
    xi = gather(iq.reshape(_B // _CH, _CH), i2)
    out = _mlp_fn(_B, _D, _H, 4096)(
        xu, xi, us.reshape(_B, 1), isel.reshape(_B, 1),
        W1[:, :_D].T, W1[:, _D:].T, b1.reshape(1, _H), W2.T, b2.reshape(1, 1))
    return out.reshape(_B)


# speedup vs baseline: 1.0057x; 1.0057x over previous
import functools

import jax
import jax.numpy as jnp
from jax import lax
from jax.experimental import pallas as pl
from jax.experimental.pallas import tpu as pltpu
from jax.experimental.pallas import tpu_sc as plsc

_B = 16384
_D = 64
_H = 128
_CH = 128
_P = 4


def _round_bits(x):
    return lax.bitcast_convert_type(
        x.astype(jnp.bfloat16).astype(jnp.float32), jnp.int32)


def _transpose_body(x0_ref, x1_ref, x2_ref, x3_ref, out_ref):
    wa = _round_bits(jnp.concatenate([x0_ref[...].T, x1_ref[...].T], axis=1))
    wb = _round_bits(jnp.concatenate([x2_ref[...].T, x3_ref[...].T], axis=1))
    out_ref[...] = wa | lax.shift_right_logical(wb, 16)


def _pack_geometry(n_rows, R):
    off = (n_rows // _P) // R * R
    grid = -(-(n_rows - (_P - 1) * off) // R)
    return off, grid


@functools.cache
def _transpose_fn(n_rows, R):
    off, grid = _pack_geometry(n_rows, R)
    ob = off // R
    return pl.pallas_call(
        _transpose_body,
        grid=(grid,),
        in_specs=[
            pl.BlockSpec((_D, R), lambda i, k=k: (0, i + k * ob))
            for k in range(_P)
        ],
        out_specs=pl.BlockSpec((R, _P * _D // 2), lambda i: (i, 0)),
        out_shape=jax.ShapeDtypeStruct((grid * R, _P * _D // 2), jnp.int32),
    )


@functools.cache
def _gather_fn(B, NC, NS, n_packed):
    NW = NC * NS
    b_per_w = B // NW
    n_ch = b_per_w // _CH
    mesh = plsc.VectorSubcoreMesh(core_axis_name="c", subcore_axis_name="s")

    @functools.partial(
        pl.kernel,
        out_type=jax.ShapeDtypeStruct((B, _P * _D // 2), jnp.int32),
        mesh=mesh,
        scratch_types=[
            pltpu.VMEM((n_ch, _CH), jnp.int32),
            pltpu.VMEM((n_ch, _CH, _P * _D // 2), jnp.int32),
            pltpu.SemaphoreType.DMA,
        ],
    )
    def gather_k(idx_hbm, tbl_hbm, x_out, idx_v, rows_v, sem):
        wid = lax.axis_index("s") * NC + lax.axis_index("c")
        base = wid * b_per_w
        pltpu.sync_copy(idx_hbm.at[pl.ds(wid * n_ch, n_ch)], idx_v)
        copies = [
            pltpu.async_copy(tbl_hbm.at[idx_v.at[c]], rows_v.at[c], sem)
            for c in range(n_ch)
        ]
        for c in range(n_ch):
            copies[c].wait()
            pltpu.sync_copy(rows_v.at[c],
                            x_out.at[pl.ds(base + c * _CH, _CH)])

    return gather_k


def _select_quarter(x_ref, sub):
    w = x_ref[...]
    hi = lax.bitcast_convert_type(w & jnp.int32(-65536), jnp.float32)
    lo = lax.bitcast_convert_type(w << 16, jnp.float32)
    a = jnp.where(sub == 0, hi[:, :_D], hi[:, _D:])
    b = jnp.where(sub == 2, lo[:, :_D], lo[:, _D:])
    return jnp.where(sub <= 1, a, b)


def _mlp_body(xu_ref, xi_ref, pu_ref, pi_ref,
              w1u_ref, w1i_ref, b1_ref, w2_ref, b2_ref, out_ref):
    ue = _select_quarter(xu_ref, pu_ref[...])
    ie = _select_quarter(xi_ref, pi_ref[...])
    h = (jnp.dot(ue, w1u_ref[...], preferred_element_type=jnp.float32)
         + jnp.dot(ie, w1i_ref[...], preferred_element_type=jnp.float32)
         + b1_ref[...])
    h = jnp.maximum(h, 0.0)
    out_ref[...] = (
        jnp.dot(h, w2_ref[...], preferred_element_type=jnp.float32) + b2_ref[...]
    )


@functools.cache
def _mlp_fn(B, D, H, blk):
    grid = B // blk
    return pl.pallas_call(
        _mlp_body,
        grid=(grid,),
        in_specs=[
            pl.BlockSpec((blk, _P * D // 2), lambda i: (i, 0)),
            pl.BlockSpec((blk, _P * D // 2), lambda i: (i, 0)),
            pl.BlockSpec((blk, 1), lambda i: (i, 0)),
            pl.BlockSpec((blk, 1), lambda i: (i, 0)),
            pl.BlockSpec((D, H), lambda i: (0, 0)),
            pl.BlockSpec((D, H), lambda i: (0, 0)),
            pl.BlockSpec((1, H), lambda i: (0, 0)),
            pl.BlockSpec((H, 1), lambda i: (0, 0)),
            pl.BlockSpec((1, 1), lambda i: (0, 0)),
        ],
        out_specs=pl.BlockSpec((blk, 1), lambda i: (i, 0)),
        out_shape=jax.ShapeDtypeStruct((B, 1), jnp.float32),
    )


def kernel(user, item, user_table, item_table, W1, b1, W2, b2):
    info = plsc.get_sparse_core_info()
    n = user_table.shape[0]
    R = 11904
    off, grid = _pack_geometry(n, R)
    tr = _transpose_fn(n, R)
    gather = _gather_fn(_B, info.num_cores, info.num_subcores, grid * R)

    def fold(i):
        sub = ((i >= off).astype(jnp.int32) + (i >= 2 * off).astype(jnp.int32)
               + (i >= 3 * off).astype(jnp.int32))
        return i - sub * off, sub

    uq, us = fold(user)
    iq, isel = fold(item)
    u2 = tr(user_table.T, user_table.T, user_table.T, user_table.T)
    xu = gather(uq.reshape(_B // _CH, _CH), u2)
    i2 = tr(item_table.T, item_table.T, item_table.T, item_table.T)
    xi = gather(iq.reshape(_B // _CH, _CH), i2)
    out = _mlp_fn(_B, _D, _H, 2048)(
        xu, xi, us.reshape(_B, 1), isel.reshape(_B, 1),
        W1[:, :_D].T, W1[:, _D:].T, b1.reshape(1, _H), W2.T, b2.reshape(1, 1))
    return out.reshape(_B)
